# baseline (speedup 1.0000x reference)
import functools

import jax
import jax.numpy as jnp
from jax import lax
from jax.experimental import pallas as pl
from jax.experimental.pallas import tpu as pltpu
from jax.experimental.pallas import tpu_sc as plsc

B = 4096
F = 26
V = 100000
D = 16

NC = 2
NS = 16
NW = NC * NS
BPW = B // NW
NIDX = F * BPW
GPR = 128

_mesh = plsc.VectorSubcoreMesh(core_axis_name="c", subcore_axis_name="s")


@functools.partial(
    pl.kernel,
    mesh=_mesh,
    out_type=jax.ShapeDtypeStruct((B,), jnp.float32),
    scratch_types=[
        pltpu.VMEM((NIDX,), jnp.int32),
        pltpu.VMEM((NIDX,), jnp.int32),
        pltpu.VMEM((2 * BPW, GPR), jnp.float32),
        pltpu.VMEM((BPW * D,), jnp.float32),
        pltpu.VMEM((BPW * D,), jnp.float32),
        pltpu.VMEM((BPW,), jnp.float32),
        pltpu.SemaphoreType.DMA,
        pltpu.SemaphoreType.DMA,
    ],
    compiler_params=pltpu.CompilerParams(needs_layout_passes=False),
)
def _fm_sc(table_hbm, idx_hbm, out_hbm,
           idx_v, sub_v, rows_v, sacc, ssacc, out_v, sem0, sem1):
    wid = lax.axis_index("s") * NC + lax.axis_index("c")
    base = wid * BPW

    pltpu.sync_copy(idx_hbm.at[wid], idx_v)

    def _prep(c, carry):
        f = c // (BPW // 16)
        sl = pl.ds(c * 16, 16)
        flat = idx_v[sl] + f * V
        idx_v[sl] = flat >> 3
        sub_v[sl] = (flat & 7) * D
        return carry

    lax.fori_loop(0, NIDX // 16, _prep, 0)

    def _zero(c, carry):
        sl = pl.ds(c * 16, 16)
        z = jnp.zeros((16,), jnp.float32)
        sacc[sl] = z
        ssacc[sl] = z
        return carry

    lax.fori_loop(0, BPW * D // 16, _zero, 0)

    def _fire(f, nb, sem):
        pltpu.async_copy(
            table_hbm.at[idx_v.at[pl.ds(f * BPW, BPW)]],
            rows_v.at[pl.ds(nb * BPW, BPW)],
            sem,
        )

    def _drain(f, nb, sem):
        pltpu.make_async_copy(
            table_hbm.at[idx_v.at[pl.ds(f * BPW, BPW)]],
            rows_v.at[pl.ds(nb * BPW, BPW)],
            sem,
        ).wait()

    _fire(0, 0, sem0)
    _fire(1, 1, sem1)

    def _compute_field(f, nb):
        for j in range(BPW // 16):
            subs = sub_v[pl.ds(f * BPW + j * 16, 16)]
            for r in range(16):
                row = nb * BPW + j * 16 + r
                off = pl.multiple_of(subs[r], D)
                v = rows_v[row, pl.ds(off, D)]
                asl = pl.ds((j * 16 + r) * D, D)
                sacc[asl] = sacc[asl] + v
                ssacc[asl] = ssacc[asl] + v * v

    def _slot(g, carry):
        for nb in range(2):
            f = 2 * g + nb
            sem = sem0 if nb == 0 else sem1
            _drain(f, nb, sem)
            _compute_field(f, nb)

            @pl.when(f + 2 < F)
            def _():
                _fire(f + 2, nb, sem)

        return carry

    lax.fori_loop(0, F // 2, _slot, 0)

    lane = lax.iota(jnp.int32, 16)
    for j in range(BPW // 16):
        res = jnp.zeros((16,), jnp.float32)
        for r in range(16):
            asl = pl.ds((j * 16 + r) * D, D)
            s = sacc[asl]
            ss = ssacc[asl]
            t = jnp.sum(s * s - ss)
            res = jnp.where(lane == r, t, res)
        out_v[pl.ds(j * 16, 16)] = 0.5 * res

    pltpu.sync_copy(out_v, out_hbm.at[pl.ds(base, BPW)])


def kernel(X, tables):
    Xp = (
        X.astype(jnp.int32)
        .reshape(NW, BPW, F)
        .transpose(0, 2, 1)
        .reshape(NW, NIDX)
    )
    t2 = tables.reshape(F * V // 8, 8 * D)
    out = _fm_sc(t2, Xp)
    return out.reshape(B, 1)


# speedup vs baseline: 1.0001x; 1.0001x over previous
import functools

import jax
import jax.numpy as jnp
from jax import lax
from jax.experimental import pallas as pl
from jax.experimental.pallas import tpu as pltpu
from jax.experimental.pallas import tpu_sc as plsc

B = 4096
F = 26
V = 100000
D = 16

NC = 2
NS = 16
NW = NC * NS
BPW = B // NW
NIDX = F * BPW
GPR = 128

_mesh = plsc.VectorSubcoreMesh(core_axis_name="c", subcore_axis_name="s")


@functools.partial(
    pl.kernel,
    mesh=_mesh,
    out_type=jax.ShapeDtypeStruct((B,), jnp.float32),
    scratch_types=[
        pltpu.VMEM((NIDX,), jnp.int32),
        pltpu.VMEM((NIDX,), jnp.int32),
        pltpu.VMEM((2 * BPW, GPR), jnp.float32),
        pltpu.VMEM((BPW * D,), jnp.float32),
        pltpu.VMEM((BPW * D,), jnp.float32),
        pltpu.VMEM((BPW,), jnp.float32),
        pltpu.SemaphoreType.DMA,
        pltpu.SemaphoreType.DMA,
    ],
    compiler_params=pltpu.CompilerParams(
        needs_layout_passes=False,
        use_tc_tiling_on_sc=True,
    ),
)
def _fm_sc(table_hbm, idx_hbm, out_hbm,
           idx_v, sub_v, rows_v, sacc, ssacc, out_v, sem0, sem1):
    wid = lax.axis_index("s") * NC + lax.axis_index("c")
    base = wid * BPW

    pltpu.sync_copy(idx_hbm.at[wid], idx_v)

    def _prep(c, carry):
        f = c // (BPW // 16)
        sl = pl.ds(c * 16, 16)
        flat = idx_v[sl] + f * V
        idx_v[sl] = flat >> 3
        sub_v[sl] = (flat & 7) * D
        return carry

    lax.fori_loop(0, NIDX // 16, _prep, 0)

    def _zero(c, carry):
        sl = pl.ds(c * 16, 16)
        z = jnp.zeros((16,), jnp.float32)
        sacc[sl] = z
        ssacc[sl] = z
        return carry

    lax.fori_loop(0, BPW * D // 16, _zero, 0)

    def _fire(f, nb, sem):
        pltpu.async_copy(
            table_hbm.at[idx_v.at[pl.ds(f * BPW, BPW)]],
            rows_v.at[pl.ds(nb * BPW, BPW)],
            sem,
        )

    def _drain(f, nb, sem):
        pltpu.make_async_copy(
            table_hbm.at[idx_v.at[pl.ds(f * BPW, BPW)]],
            rows_v.at[pl.ds(nb * BPW, BPW)],
            sem,
        ).wait()

    _fire(0, 0, sem0)
    _fire(1, 1, sem1)

    def _compute_field(f, nb):
        for j in range(BPW // 16):
            subs = sub_v[pl.ds(f * BPW + j * 16, 16)]
            for r in range(16):
                row = nb * BPW + j * 16 + r
                off = pl.multiple_of(subs[r], D)
                v = rows_v[row, pl.ds(off, D)]
                asl = pl.ds((j * 16 + r) * D, D)
                sacc[asl] = sacc[asl] + v
                ssacc[asl] = ssacc[asl] + v * v

    def _slot(g, carry):
        for nb in range(2):
            f = 2 * g + nb
            sem = sem0 if nb == 0 else sem1
            _drain(f, nb, sem)
            _compute_field(f, nb)

            @pl.when(f + 2 < F)
            def _():
                _fire(f + 2, nb, sem)

        return carry

    lax.fori_loop(0, F // 2, _slot, 0)

    lane = lax.iota(jnp.int32, 16)
    for j in range(BPW // 16):
        res = jnp.zeros((16,), jnp.float32)
        for r in range(16):
            asl = pl.ds((j * 16 + r) * D, D)
            s = sacc[asl]
            ss = ssacc[asl]
            t = jnp.sum(s * s - ss)
            res = jnp.where(lane == r, t, res)
        out_v[pl.ds(j * 16, 16)] = 0.5 * res

    pltpu.sync_copy(out_v, out_hbm.at[pl.ds(base, BPW)])


def kernel(X, tables):
    Xp = (
        X.astype(jnp.int32)
        .reshape(NW, BPW, F)
        .transpose(0, 2, 1)
        .reshape(NW, NIDX)
    )
    t2 = tables.reshape(F * V // 8, 8 * D)
    out = _fm_sc(t2, Xp)
    return out.reshape(B, 1)


# speedup vs baseline: 1.0476x; 1.0475x over previous
import functools

import jax
import jax.numpy as jnp
from jax import lax
from jax.experimental import pallas as pl
from jax.experimental.pallas import tpu as pltpu
from jax.experimental.pallas import tpu_sc as plsc

B = 4096
F = 26
V = 100000
D = 16

NC = 2
NS = 16
NW = NC * NS
BPW = B // NW
NIDX = F * BPW

_mesh = plsc.VectorSubcoreMesh(core_axis_name="c", subcore_axis_name="s")


@functools.partial(
    pl.kernel,
    mesh=_mesh,
    out_type=jax.ShapeDtypeStruct((B,), jnp.float32),
    scratch_types=[
        pltpu.VMEM((NIDX,), jnp.int32),
        pltpu.VMEM((NIDX, D), jnp.float32),
        pltpu.VMEM((BPW,), jnp.float32),
        pltpu.SemaphoreType.DMA,
    ],
    compiler_params=pltpu.CompilerParams(
        needs_layout_passes=False,
        use_tc_tiling_on_sc=False,
    ),
)
def _fm_sc(table_hbm, idx_hbm, out_hbm, idx_v, rows_v, out_v, sem):
    wid = lax.axis_index("s") * NC + lax.axis_index("c")
    base = wid * BPW

    pltpu.sync_copy(idx_hbm.at[wid], idx_v)

    def _add_off(i, carry):
        f = i // (BPW // 16)
        sl = pl.ds(i * 16, 16)
        idx_v[sl] = idx_v[sl] + f * V
        return carry

    lax.fori_loop(0, NIDX // 16, _add_off, 0)

    copies = [
        pltpu.async_copy(
            table_hbm.at[idx_v.at[pl.ds(f * BPW, BPW)]],
            rows_v.at[pl.ds(f * BPW, BPW)],
            sem,
        )
        for f in range(F)
    ]
    for cp in copies:
        cp.wait()

    lane = lax.iota(jnp.int32, 16)

    def _group(g, carry):
        b0 = g * 16
        res = jnp.zeros((16,), jnp.float32)
        for j in range(16):
            s = jnp.zeros((D,), jnp.float32)
            ss = jnp.zeros((D,), jnp.float32)
            for f in range(F):
                v = rows_v[f * BPW + b0 + j, :]
                s = s + v
                ss = ss + v * v
            r = jnp.sum(s * s - ss)
            res = jnp.where(lane == j, r, res)
        out_v[pl.ds(b0, 16)] = 0.5 * res
        return carry

    lax.fori_loop(0, BPW // 16, _group, 0)

    pltpu.sync_copy(out_v, out_hbm.at[pl.ds(base, BPW)])


def kernel(X, tables):
    Xp = (
        X.astype(jnp.int32)
        .reshape(NW, BPW, F)
        .transpose(0, 2, 1)
        .reshape(NW, NIDX)
    )
    one = (X[0, 0] * 0 + 1).astype(jnp.float32)
    out = _fm_sc(tables.reshape(F * V, D) * one, Xp)
    return out.reshape(B, 1)
